# Initial kernel scaffold; baseline (speedup 1.0000x reference)
import jax
import jax.numpy as jnp
from jax.experimental import pallas as pl


def kernel(x_numerical, x_categorical, tables, W1, b1, W2, b2, W3, b3, Wp, bp):
    raise NotImplementedError("write your pallas kernel here")


# speedup vs baseline: 2.0162x; 2.0162x over previous
import functools

import jax
import jax.numpy as jnp
from jax import lax
from jax.experimental import pallas as pl
from jax.experimental.pallas import tpu as pltpu
from jax.experimental.pallas import tpu_sc as plsc

B = 16384
F = 26
V = 100000
D = 32
NUM = 13

NC = 2
NS = 16
NW = NC * NS

BF = B * F
IDX_COLS = 128
IDX_ROWS = BF // IDX_COLS
ROWS_PER_W = IDX_ROWS // NW
MACRO = 8
N_MACRO = ROWS_PER_W // MACRO
CHUNK = MACRO * IDX_COLS


def _gather_kernel(tab_hbm, idx_hbm, out_hbm, idx_v, rows_v, sem):
    wid = lax.axis_index("s") * NC + lax.axis_index("c")
    pltpu.sync_copy(idx_hbm.at[pl.ds(wid * ROWS_PER_W, ROWS_PER_W)], idx_v)

    out_base = wid * (ROWS_PER_W * IDX_COLS)

    def body(c, carry):
        handles = []
        for k in range(MACRO):
            handles.append(pltpu.async_copy(
                tab_hbm.at[idx_v.at[c * MACRO + k]],
                rows_v.at[pl.ds(k * IDX_COLS, IDX_COLS)],
                sem))
        for h in handles:
            h.wait()
        pltpu.sync_copy(rows_v, out_hbm.at[pl.ds(out_base + c * CHUNK, CHUNK)])
        return carry

    lax.fori_loop(0, N_MACRO, body, 0, unroll=False)


@functools.cache
def _sc_gather():
    return pl.kernel(
        _gather_kernel,
        out_type=jax.ShapeDtypeStruct((BF, D), jnp.float32),
        mesh=plsc.VectorSubcoreMesh(core_axis_name="c", subcore_axis_name="s"),
        scratch_types=[
            pltpu.VMEM((ROWS_PER_W, IDX_COLS), jnp.int32),
            pltpu.VMEM((CHUNK, D), jnp.float32),
            pltpu.SemaphoreType.DMA,
        ],
        compiler_params=pltpu.CompilerParams(use_tc_tiling_on_sc=False),
    )


BB = 1024


def _mlp_kernel(h_ref, xn_ref, w1_ref, b1_ref, w2_ref, b2_ref, w3_ref,
                b3_ref, wpd_ref, wpw_ref, bp_ref, o_ref):
    h = h_ref[...]
    h1 = jnp.dot(h, w1_ref[...], preferred_element_type=jnp.float32)
    h1 = h1 + b1_ref[...]
    h1 = jnp.where(h1 > 0, h1, 0.01 * h1)
    h2 = jnp.dot(h1, w2_ref[...], preferred_element_type=jnp.float32)
    h2 = h2 + b2_ref[...]
    h2 = jnp.where(h2 > 0, h2, 0.01 * h2)
    h3 = jnp.dot(h2, w3_ref[...], preferred_element_type=jnp.float32)
    h3 = jnp.tanh(h3 + b3_ref[...])
    z = (jnp.dot(h3, wpd_ref[...], preferred_element_type=jnp.float32)
         + jnp.dot(xn_ref[...], wpw_ref[...], preferred_element_type=jnp.float32)
         + bp_ref[...])
    o_ref[...] = 1.0 / (1.0 + jnp.exp(-z))


def _mlp(h, x_numerical, W1, b1, W2, b2, W3, b3, Wp, bp):
    wpd = Wp[:64]
    wpw = Wp[64:]
    grid = (B // BB,)
    fixed = lambda i: (0, 0)
    return pl.pallas_call(
        _mlp_kernel,
        grid=grid,
        in_specs=[
            pl.BlockSpec((BB, F * D), lambda i: (i, 0)),
            pl.BlockSpec((BB, NUM), lambda i: (i, 0)),
            pl.BlockSpec((F * D, 256), fixed),
            pl.BlockSpec((1, 256), fixed),
            pl.BlockSpec((256, 128), fixed),
            pl.BlockSpec((1, 128), fixed),
            pl.BlockSpec((128, 64), fixed),
            pl.BlockSpec((1, 64), fixed),
            pl.BlockSpec((64, 1), fixed),
            pl.BlockSpec((NUM, 1), fixed),
            pl.BlockSpec((1, 1), fixed),
        ],
        out_specs=pl.BlockSpec((BB, 1), lambda i: (i, 0)),
        out_shape=jax.ShapeDtypeStruct((B, 1), jnp.float32),
    )(h, x_numerical, W1, b1.reshape(1, 256), W2, b2.reshape(1, 128),
      W3, b3.reshape(1, 64), wpd, wpw, bp.reshape(1, 1))


def kernel(x_numerical, x_categorical, tables, W1, b1, W2, b2, W3, b3, Wp, bp):
    idx = (x_categorical + jnp.arange(F, dtype=jnp.int32) * V)
    idx = idx.reshape(IDX_ROWS, IDX_COLS)
    tab = tables.reshape(F * V, D)
    h_flat = _sc_gather()(tab, idx)
    h = h_flat.reshape(B, F * D)
    return _mlp(h, x_numerical, W1, b1, W2, b2, W3, b3, Wp, bp)


# speedup vs baseline: 5.9188x; 2.9357x over previous
import functools

import jax
import jax.numpy as jnp
from jax import lax
from jax.experimental import pallas as pl
from jax.experimental.pallas import tpu as pltpu
from jax.experimental.pallas import tpu_sc as plsc

B = 16384
F = 26
V = 100000
D = 32
NUM = 13

NC = 2
NS = 16
NW = NC * NS

ROWS = F * D
ROWS_PER_W = ROWS // NW
OCHUNK = 2048
N_OCHUNK = B // OCHUNK


def _gather_kernel(tab_hbm, xt_hbm, out_hbm, row_v, idx_v, out_v):
    wid = lax.axis_index("s") * NC + lax.axis_index("c")
    base_r = wid * ROWS_PER_W

    def row_body(j, _):
        r = base_r + j
        f = r >> 5

        @pl.when((j == 0) | ((r & 31) == 0))
        def _load_idx():
            pltpu.sync_copy(xt_hbm.at[f], idx_v)

        pltpu.sync_copy(tab_hbm.at[r], row_v)

        def chunk_body(c, _):
            cbase = c * OCHUNK

            def vec_body(i, _):
                ids = idx_v[pl.ds(cbase + i * 16, 16)]
                out_v[pl.ds(i * 16, 16)] = plsc.load_gather(row_v, [ids])
                return 0

            lax.fori_loop(0, OCHUNK // 16, vec_body, 0, unroll=8)
            pltpu.sync_copy(out_v, out_hbm.at[r, pl.ds(cbase, OCHUNK)])
            return 0

        lax.fori_loop(0, N_OCHUNK, chunk_body, 0)
        return 0

    lax.fori_loop(0, ROWS_PER_W, row_body, 0)


@functools.cache
def _sc_gather():
    return pl.kernel(
        _gather_kernel,
        out_type=jax.ShapeDtypeStruct((ROWS, B), jnp.float32),
        mesh=plsc.VectorSubcoreMesh(core_axis_name="c", subcore_axis_name="s"),
        scratch_types=[
            pltpu.VMEM((V,), jnp.float32),
            pltpu.VMEM((B,), jnp.int32),
            pltpu.VMEM((OCHUNK,), jnp.float32),
        ],
        compiler_params=pltpu.CompilerParams(
            use_tc_tiling_on_sc=True,
            needs_layout_passes=False,
        ),
    )


BB = 1024


def _mlp_kernel(ht_ref, xn_ref, w1_ref, b1_ref, w2_ref, b2_ref, w3_ref,
                b3_ref, wpd_ref, wpw_ref, bp_ref, o_ref):
    h1 = lax.dot_general(ht_ref[...], w1_ref[...], (((0,), (0,)), ((), ())),
                         preferred_element_type=jnp.float32)
    h1 = h1 + b1_ref[...]
    h1 = jnp.where(h1 > 0, h1, 0.01 * h1)
    h2 = jnp.dot(h1, w2_ref[...], preferred_element_type=jnp.float32)
    h2 = h2 + b2_ref[...]
    h2 = jnp.where(h2 > 0, h2, 0.01 * h2)
    h3 = jnp.dot(h2, w3_ref[...], preferred_element_type=jnp.float32)
    h3 = jnp.tanh(h3 + b3_ref[...])
    z = (jnp.dot(h3, wpd_ref[...], preferred_element_type=jnp.float32)
         + jnp.dot(xn_ref[...], wpw_ref[...], preferred_element_type=jnp.float32)
         + bp_ref[...])
    o_ref[...] = 1.0 / (1.0 + jnp.exp(-z))


def _mlp(ht, x_numerical, W1, b1, W2, b2, W3, b3, Wp, bp):
    wpd = Wp[:64]
    wpw = Wp[64:]
    grid = (B // BB,)
    fixed = lambda i: (0, 0)
    return pl.pallas_call(
        _mlp_kernel,
        grid=grid,
        in_specs=[
            pl.BlockSpec((F * D, BB), lambda i: (0, i)),
            pl.BlockSpec((BB, NUM), lambda i: (i, 0)),
            pl.BlockSpec((F * D, 256), fixed),
            pl.BlockSpec((1, 256), fixed),
            pl.BlockSpec((256, 128), fixed),
            pl.BlockSpec((1, 128), fixed),
            pl.BlockSpec((128, 64), fixed),
            pl.BlockSpec((1, 64), fixed),
            pl.BlockSpec((64, 1), fixed),
            pl.BlockSpec((NUM, 1), fixed),
            pl.BlockSpec((1, 1), fixed),
        ],
        out_specs=pl.BlockSpec((BB, 1), lambda i: (i, 0)),
        out_shape=jax.ShapeDtypeStruct((B, 1), jnp.float32),
    )(ht, x_numerical, W1, b1.reshape(1, 256), W2, b2.reshape(1, 128),
      W3, b3.reshape(1, 64), wpd, wpw, bp.reshape(1, 1))


def kernel(x_numerical, x_categorical, tables, W1, b1, W2, b2, W3, b3, Wp, bp):
    tabt = tables.transpose(0, 2, 1).reshape(ROWS, V)
    xt = x_categorical.T
    ht = _sc_gather()(tabt, xt)
    return _mlp(ht, x_numerical, W1, b1, W2, b2, W3, b3, Wp, bp)


# speedup vs baseline: 10.8574x; 1.8344x over previous
import functools

import jax
import jax.numpy as jnp
from jax import lax
from jax.experimental import pallas as pl
from jax.experimental.pallas import tpu as pltpu
from jax.experimental.pallas import tpu_sc as plsc

B = 16384
F = 26
V = 100000
D = 32
NUM = 13

NC = 2
NS = 16
NW = NC * NS

ROWS = F * D
ROWS_PER_W = ROWS // NW
OCHUNK = 4096
N_OCHUNK = B // OCHUNK


def _gather_kernel(tab_hbm, xt_hbm, out_hbm, row_v, idx_v, out_v, sems):
    wid = lax.axis_index("s") * NC + lax.axis_index("c")
    base_r = wid * ROWS_PER_W

    def row_body(j, _):
        r = base_r + j
        f = r >> 5

        @pl.when((j == 0) | ((r & 31) == 0))
        def _load_idx():
            pltpu.sync_copy(xt_hbm.at[f], idx_v)

        pltpu.sync_copy(tab_hbm.at[r], row_v)

        def chunk_body(c, _):
            cbase = c * OCHUNK
            slot = lax.rem(c, 2)
            dst = out_hbm.at[r, pl.ds(cbase, OCHUNK)]

            @pl.when(c >= 2)
            def _drain():
                pltpu.make_async_copy(out_v.at[slot], dst, sems.at[slot]).wait()

            @plsc.parallel_loop(0, OCHUNK // 16, unroll=8)
            def vec_body(i):
                ids = idx_v[pl.ds(cbase + i * 16, 16)]
                out_v[slot, pl.ds(i * 16, 16)] = plsc.load_gather(row_v, [ids])

            pltpu.async_copy(out_v.at[slot], dst, sems.at[slot])
            return 0

        lax.fori_loop(0, N_OCHUNK, chunk_body, 0)
        for slot in range(2):
            pltpu.make_async_copy(
                out_v.at[slot], out_hbm.at[r, pl.ds(0, OCHUNK)], sems.at[slot]
            ).wait()
        return 0

    lax.fori_loop(0, ROWS_PER_W, row_body, 0)


@functools.cache
def _sc_gather():
    return pl.kernel(
        _gather_kernel,
        out_type=jax.ShapeDtypeStruct((ROWS, B), jnp.float32),
        mesh=plsc.VectorSubcoreMesh(core_axis_name="c", subcore_axis_name="s"),
        scratch_types=[
            pltpu.VMEM((V,), jnp.float32),
            pltpu.VMEM((B,), jnp.int32),
            pltpu.VMEM((2, OCHUNK), jnp.float32),
            pltpu.SemaphoreType.DMA((2,)),
        ],
        compiler_params=pltpu.CompilerParams(
            use_tc_tiling_on_sc=True,
            needs_layout_passes=False,
        ),
    )


BB = 1024


def _mlp_kernel(ht_ref, xn_ref, w1_ref, b1_ref, w2_ref, b2_ref, w3_ref,
                b3_ref, wpd_ref, wpw_ref, bp_ref, o_ref):
    h1 = lax.dot_general(ht_ref[...], w1_ref[...], (((0,), (0,)), ((), ())),
                         preferred_element_type=jnp.float32)
    h1 = h1 + b1_ref[...]
    h1 = jnp.where(h1 > 0, h1, 0.01 * h1)
    h2 = jnp.dot(h1, w2_ref[...], preferred_element_type=jnp.float32)
    h2 = h2 + b2_ref[...]
    h2 = jnp.where(h2 > 0, h2, 0.01 * h2)
    h3 = jnp.dot(h2, w3_ref[...], preferred_element_type=jnp.float32)
    h3 = jnp.tanh(h3 + b3_ref[...])
    z = (jnp.dot(h3, wpd_ref[...], preferred_element_type=jnp.float32)
         + jnp.dot(xn_ref[...], wpw_ref[...], preferred_element_type=jnp.float32)
         + bp_ref[...])
    o_ref[...] = 1.0 / (1.0 + jnp.exp(-z))


def _mlp(ht, x_numerical, W1, b1, W2, b2, W3, b3, Wp, bp):
    wpd = Wp[:64]
    wpw = Wp[64:]
    grid = (B // BB,)
    fixed = lambda i: (0, 0)
    return pl.pallas_call(
        _mlp_kernel,
        grid=grid,
        in_specs=[
            pl.BlockSpec((F * D, BB), lambda i: (0, i)),
            pl.BlockSpec((BB, NUM), lambda i: (i, 0)),
            pl.BlockSpec((F * D, 256), fixed),
            pl.BlockSpec((1, 256), fixed),
            pl.BlockSpec((256, 128), fixed),
            pl.BlockSpec((1, 128), fixed),
            pl.BlockSpec((128, 64), fixed),
            pl.BlockSpec((1, 64), fixed),
            pl.BlockSpec((64, 1), fixed),
            pl.BlockSpec((NUM, 1), fixed),
            pl.BlockSpec((1, 1), fixed),
        ],
        out_specs=pl.BlockSpec((BB, 1), lambda i: (i, 0)),
        out_shape=jax.ShapeDtypeStruct((B, 1), jnp.float32),
    )(ht, x_numerical, W1, b1.reshape(1, 256), W2, b2.reshape(1, 128),
      W3, b3.reshape(1, 64), wpd, wpw, bp.reshape(1, 1))


def kernel(x_numerical, x_categorical, tables, W1, b1, W2, b2, W3, b3, Wp, bp):
    tabt = tables.transpose(0, 2, 1).reshape(ROWS, V)
    xt = x_categorical.T
    ht = _sc_gather()(tabt, xt)
    return _mlp(ht, x_numerical, W1, b1, W2, b2, W3, b3, Wp, bp)
